# baseline (speedup 1.0000x reference)
import functools

import jax
import jax.numpy as jnp
from jax import lax
from jax.experimental import pallas as pl
from jax.experimental.pallas import tpu as pltpu
from jax.experimental.pallas import tpu_sc as plsc

_MAX_SIZE = 4096
_N_PROTO = 51
_FEAT_DIM = 128
_N_REL = 16384
_INV_THRESH = 1.0 / 0.9
_LN9 = 2.1972245773362196

_NC = 2
_NS = 16
_NW = _NC * _NS
_CHUNK = _N_REL // _NW
_L = 16
_NG = _CHUNK // _L


def _tec_body(feature_hbm, logits_hbm, slot_hbm, mem_ref,
              logits_v, slot_v, sel_all, targ_all, acc_v, row_v,
              sem_a, sem_b):
  wid = lax.axis_index("s") * _NC + lax.axis_index("c")
  base = wid * _CHUNK
  cp_slot = pltpu.async_copy(slot_hbm.at[pl.ds(base, _CHUNK)], slot_v, sem_a)
  cp_log = pltpu.async_copy(
      logits_hbm.at[:, pl.ds(base, _CHUNK)], logits_v, sem_b)
  cp_slot.wait()
  cp_log.wait()

  def group(g, carry):
    for u in range(2):
      off = g * (2 * _L) + u * _L
      sl = pl.ds(off, _L)
      m = logits_v[0, sl]
      m2 = jnp.full((_L,), -3.0e38, jnp.float32)
      amax = jnp.zeros((_L,), jnp.int32)
      for c in range(1, _N_PROTO):
        z = logits_v[c, sl]
        gt = z > m
        amax = jnp.where(gt, c, amax)
        m2 = jnp.maximum(m2, jnp.minimum(z, m))
        m = jnp.maximum(m, z)
      maybe = jnp.where(m - m2 > _LN9, 1.0, 0.0)
      mbv = maybe[0]
      for i in range(1, _L):
        mbv = mbv + maybe[i]

      sel_all[sl] = jnp.zeros((_L,), jnp.float32)

      @pl.when(mbv > 0.0)
      def _exact():
        ssum = jnp.zeros((_L,), jnp.float32)
        for c in range(_N_PROTO):
          ssum = ssum + jnp.exp(logits_v[c, sl] - m)
        selv = jnp.where(ssum < _INV_THRESH, 1.0, 0.0)
        sel_all[sl] = selv
        targ_all[sl] = amax * _MAX_SIZE + slot_v[sl]
        acc_v[...] = acc_v[...] + selv

    return carry

  acc_v[...] = jnp.zeros((_L,), jnp.float32)
  lax.fori_loop(0, _NG // 2, group, 0)
  avals = acc_v[...]
  cnt = avals[0]
  for i in range(1, _L):
    cnt = cnt + avals[i]

  @pl.when(cnt > 0.0)
  def _scatter_rare():
    def wgroup(g, carry):
      off = g * _L
      sl = pl.ds(off, _L)
      selv = sel_all[sl]
      targ = targ_all[sl]
      for i in range(_L):
        @pl.when(selv[i] > 0.0)
        def _write():
          pltpu.sync_copy(feature_hbm.at[pl.ds(base + off + i, 1), :], row_v)
          pltpu.sync_copy(row_v, mem_ref.at[pl.ds(targ[i], 1), :])
      return carry

    lax.fori_loop(0, _NG, wgroup, 0)


_mesh = plsc.VectorSubcoreMesh(core_axis_name="c", subcore_axis_name="s")

_scatter = pl.kernel(
    _tec_body,
    out_type=(),
    mesh=_mesh,
    scratch_types=[
        pltpu.VMEM((_N_PROTO, _CHUNK), jnp.float32),
        pltpu.VMEM((_CHUNK,), jnp.int32),
        pltpu.VMEM((_CHUNK,), jnp.float32),
        pltpu.VMEM((_CHUNK,), jnp.int32),
        pltpu.VMEM((_L,), jnp.float32),
        pltpu.VMEM((1, _FEAT_DIM), jnp.float32),
        pltpu.SemaphoreType.DMA,
        pltpu.SemaphoreType.DMA,
    ],
    name="memory_banks_scatter",
)


def kernel(mem, feature, rel_logits, slot_idx):
  logits_t = rel_logits.T
  mem_ref = jax.new_ref(mem)
  _scatter(feature, logits_t, slot_idx, mem_ref)
  return mem_ref[...]
